# baseline (speedup 1.0000x reference)
import functools

import jax
import jax.numpy as jnp
from jax import lax
from jax.experimental import pallas as pl
from jax.experimental.pallas import tpu as pltpu
from jax.experimental.pallas import tpu_sc as plsc

N = 10000
E = 320000
D = 128
R = 8
NRF = N * R

NC = 2
NS = 16
NW = NC * NS
EPW = E // NW
EB = 125
NBLK = EPW // EB
CB = 8
SB = 2000
NSBLK = EPW // SB

_mesh = functools.partial(
    plsc.VectorSubcoreMesh, core_axis_name="c", subcore_axis_name="s")


def _fill_rows(ref, nrows, ncols, value, dtype):
    v = jnp.full((16,), value, dtype)

    def body(i, _):
        for cc in range(ncols // 16):
            ref[i, pl.ds(cc * 16, 16)] = v
        return ()

    lax.fori_loop(0, nrows, body, ())


def _h_body(x_ref, w_ref, h_ref):
    h_ref[0] = jnp.dot(x_ref[...], w_ref[0],
                       preferred_element_type=jnp.float32)


def _make_h(x, W):
    nb = 2000
    return pl.pallas_call(
        _h_body,
        grid=(N // nb, R),
        in_specs=[
            pl.BlockSpec((nb, D), lambda n, r: (n, 0)),
            pl.BlockSpec((1, D, D), lambda n, r: (r, 0, 0)),
        ],
        out_specs=pl.BlockSpec((1, nb, D), lambda n, r: (r, n, 0)),
        out_shape=jax.ShapeDtypeStruct((R, N, D), jnp.float32),
    )(x, W)


def _count_body(zero_hbm, dst_hbm, typ_hbm, cpart_hbm,
                hist_v, dst_v, typ_v):
    c = lax.axis_index("c")
    s = lax.axis_index("s")
    wid = c * NS + s

    pltpu.sync_copy(zero_hbm, hist_v)

    ones = jnp.full((16,), 1.0, jnp.float32)

    def blk(j, _):
        off = wid * EPW + j * SB
        pltpu.sync_copy(dst_hbm.at[pl.ds(off, SB)], dst_v)
        pltpu.sync_copy(typ_hbm.at[pl.ds(off, SB)], typ_v)

        def grp(i, _):
            sl = pl.ds(i * 16, 16)
            cidx = dst_v[sl] * R + typ_v[sl]
            plsc.addupdate_scatter(hist_v, [cidx], ones)
            return ()

        lax.fori_loop(0, SB // 16, grp, ())
        return ()

    lax.fori_loop(0, NSBLK, blk, ())
    pltpu.sync_copy(hist_v, cpart_hbm.at[wid])


def _make_counts(dst, typ):
    kern = pl.kernel(
        _count_body,
        out_type=jax.ShapeDtypeStruct((NW, NRF), jnp.float32),
        mesh=_mesh(),
        scratch_types=[
            pltpu.VMEM((NRF,), jnp.float32),
            pltpu.VMEM((SB,), jnp.int32),
            pltpu.VMEM((SB,), jnp.int32),
        ],
        compiler_params=pltpu.CompilerParams(needs_layout_passes=False),
    )
    return kern(jnp.zeros((NRF,), jnp.float32), dst, typ)


def _inv_body(cp_ref, inv_ref):
    i = pl.program_id(0)

    @pl.when(i == 0)
    def _():
        inv_ref[...] = jnp.zeros_like(inv_ref)

    inv_ref[...] += jnp.sum(cp_ref[...], axis=0)

    @pl.when(i == pl.num_programs(0) - 1)
    def _():
        inv_ref[...] = 1.0 / jnp.maximum(inv_ref[...], 1.0)


def _make_inv(cpart):
    nw_b = 8
    cp3 = cpart.reshape(NW, NRF // 128, 128)
    inv2d = pl.pallas_call(
        _inv_body,
        grid=(NW // nw_b,),
        in_specs=[pl.BlockSpec((nw_b, NRF // 128, 128), lambda i: (i, 0, 0))],
        out_specs=pl.BlockSpec((NRF // 128, 128), lambda i: (0, 0)),
        out_shape=jax.ShapeDtypeStruct((NRF // 128, 128), jnp.float32),
    )(cp3)
    return inv2d.reshape(NRF)


def _scale_body(inv_hbm, src_hbm, dst_hbm, typ_hbm, gidx_hbm, sval_hbm,
                inv_v, src_v, dst_v, typ_v, gidx_v, sval_v):
    c = lax.axis_index("c")
    s = lax.axis_index("s")
    wid = c * NS + s

    pltpu.sync_copy(inv_hbm, inv_v)

    def blk(j, _):
        off = wid * EPW + j * SB
        pltpu.sync_copy(src_hbm.at[pl.ds(off, SB)], src_v)
        pltpu.sync_copy(dst_hbm.at[pl.ds(off, SB)], dst_v)
        pltpu.sync_copy(typ_hbm.at[pl.ds(off, SB)], typ_v)

        @plsc.parallel_loop(0, SB // 16, unroll=5)
        def _grp(i):
            sl = pl.ds(i * 16, 16)
            gidx_v[sl] = typ_v[sl] * N + src_v[sl]
            cix = dst_v[sl] * R + typ_v[sl]
            sval_v[sl] = plsc.load_gather(inv_v, [cix])
        pltpu.sync_copy(gidx_v, gidx_hbm.at[pl.ds(off, SB)])
        pltpu.sync_copy(sval_v, sval_hbm.at[pl.ds(off, SB)])
        return ()

    lax.fori_loop(0, NSBLK, blk, ())


def _make_scale(inv, src, dst, typ):
    kern = pl.kernel(
        _scale_body,
        out_type=(jax.ShapeDtypeStruct((E,), jnp.int32),
                  jax.ShapeDtypeStruct((E,), jnp.float32)),
        mesh=_mesh(),
        scratch_types=[
            pltpu.VMEM((NRF,), jnp.float32),
            pltpu.VMEM((SB,), jnp.int32),
            pltpu.VMEM((SB,), jnp.int32),
            pltpu.VMEM((SB,), jnp.int32),
            pltpu.VMEM((SB,), jnp.int32),
            pltpu.VMEM((SB,), jnp.float32),
        ],
        compiler_params=pltpu.CompilerParams(needs_layout_passes=False),
    )
    return kern(inv, src, dst, typ)


def _main_body(h_hbm, gidx_hbm, dst_hbm, sval_hbm, part_hbm,
               acc_sh, gidx_c, dst_c, sval_c, hbuf_a, hbuf_b,
               gsem_a, gsem_b, ssem_a, ssem_b):
    c = lax.axis_index("c")
    s = lax.axis_index("s")
    wid = c * NS + s

    _fill_rows(hbuf_a, 40, D, 0.0, jnp.float32)
    nz_s = jnp.where(s < 10, 16, 15)

    def zero_blk(k, _):
        pltpu.sync_copy(hbuf_a.at[pl.ds(0, 40)],
                        acc_sh.at[pl.ds((s + k * NS) * 40, 40)])
        return ()

    lax.fori_loop(0, nz_s, zero_blk, ())
    plsc.subcore_barrier()

    hbufs = (hbuf_a, hbuf_b)
    gsems = (gsem_a, gsem_b)
    ssems = (ssem_a, ssem_b)

    def scale_rows(hbuf, bb):
        @plsc.parallel_loop(0, EB, unroll=25)
        def _srow(e):
            svec = plsc.load_gather(
                sval_c, [jnp.full((16,), bb, jnp.int32),
                         jnp.full((16,), e, jnp.int32)])
            for cc in range(D // 16):
                csl = pl.ds(cc * 16, 16)
                hbuf[e, csl] = hbuf[e, csl] * svec

    def chunk(ch, _):
        rows = pl.ds(ch * CB, CB)
        pltpu.sync_copy(gidx_hbm.at[wid].at[rows], gidx_c)
        pltpu.sync_copy(dst_hbm.at[wid].at[rows], dst_c)
        pltpu.sync_copy(sval_hbm.at[wid].at[rows], sval_c)

        pltpu.async_copy(h_hbm.at[gidx_c.at[0]], hbufs[0], gsems[0])
        for b in range(CB):
            p = b % 2
            q = 1 - p
            if b + 1 < CB:
                if b >= 1:
                    pltpu.make_async_copy(hbufs[q], acc_sh.at[dst_c.at[b - 1]],
                                          ssems[q]).wait()
                pltpu.async_copy(h_hbm.at[gidx_c.at[b + 1]], hbufs[q],
                                 gsems[q])
            pltpu.make_async_copy(h_hbm.at[gidx_c.at[b]], hbufs[p],
                                  gsems[p]).wait()
            scale_rows(hbufs[p], b)
            if b + 1 < CB:
                pltpu.async_copy(hbufs[p], acc_sh.at[dst_c.at[b]], ssems[p],
                                 add=True)
            else:
                pltpu.sync_copy(hbufs[p], acc_sh.at[dst_c.at[b]], add=True)
        pltpu.make_async_copy(hbufs[CB % 2], acc_sh.at[dst_c.at[CB - 2]],
                              ssems[CB % 2]).wait()
        return ()

    lax.fori_loop(0, NBLK // CB, chunk, ())
    plsc.subcore_barrier()

    def out_blk(k, _):
        rows = pl.ds((s + k * NS) * 40, 40)
        pltpu.sync_copy(acc_sh.at[rows], hbuf_b.at[pl.ds(0, 40)])
        pltpu.sync_copy(hbuf_b.at[pl.ds(0, 40)], part_hbm.at[c].at[rows])
        return ()

    lax.fori_loop(0, nz_s, out_blk, ())


def _make_parts(h2, gidx, dst, sval):
    kern = pl.kernel(
        _main_body,
        out_type=jax.ShapeDtypeStruct((NC, N, D), jnp.float32),
        mesh=_mesh(),
        scratch_types=[
            pltpu.VMEM_SHARED((N, D), jnp.float32),
            pltpu.VMEM((CB, EB), jnp.int32),
            pltpu.VMEM((CB, EB), jnp.int32),
            pltpu.VMEM((CB, EB), jnp.float32),
            pltpu.VMEM((EB, D), jnp.float32),
            pltpu.VMEM((EB, D), jnp.float32),
            pltpu.SemaphoreType.DMA,
            pltpu.SemaphoreType.DMA,
            pltpu.SemaphoreType.DMA,
            pltpu.SemaphoreType.DMA,
        ],
        compiler_params=pltpu.CompilerParams(needs_layout_passes=False),
    )
    g3 = gidx.reshape(NW, NBLK, EB)
    d3 = dst.reshape(NW, NBLK, EB)
    v3 = sval.reshape(NW, NBLK, EB)
    return kern(h2, g3, d3, v3)


def _final_body(x_ref, root_ref, bias_ref, part_ref, out_ref):
    acc = jnp.dot(x_ref[...], root_ref[...],
                  preferred_element_type=jnp.float32)
    acc = acc + bias_ref[...] + part_ref[0] + part_ref[1]
    out_ref[...] = jnp.maximum(acc, 0.0)


def _make_out(x, root, bias, part):
    nb = 2000
    return pl.pallas_call(
        _final_body,
        grid=(N // nb,),
        in_specs=[
            pl.BlockSpec((nb, D), lambda n: (n, 0)),
            pl.BlockSpec((D, D), lambda n: (0, 0)),
            pl.BlockSpec((1, D), lambda n: (0, 0)),
            pl.BlockSpec((NC, nb, D), lambda n: (0, n, 0)),
        ],
        out_specs=pl.BlockSpec((nb, D), lambda n: (n, 0)),
        out_shape=jax.ShapeDtypeStruct((N, D), jnp.float32),
    )(x, root, bias.reshape(1, D), part)


@jax.jit
def kernel(x, edge_index, edge_type, W, root, bias):
    src = edge_index[0]
    dst = edge_index[1]
    typ = edge_type

    h = _make_h(x, W)
    h2 = h.reshape(R * N, D)
    cpart = _make_counts(dst, typ)
    inv = _make_inv(cpart)
    gidx, sval = _make_scale(inv, src, dst, typ)
    part = _make_parts(h2, gidx, dst, sval)
    return _make_out(x, root, bias, part)


# speedup vs baseline: 1.0135x; 1.0135x over previous
import functools

import jax
import jax.numpy as jnp
from jax import lax
from jax.experimental import pallas as pl
from jax.experimental.pallas import tpu as pltpu
from jax.experimental.pallas import tpu_sc as plsc

N = 10000
E = 320000
D = 128
R = 8
NRF = N * R

NC = 2
NS = 16
NW = NC * NS
EPW = E // NW
EB = 125
NBLK = EPW // EB
CB = 8
SB = 2000
NSBLK = EPW // SB

_mesh = functools.partial(
    plsc.VectorSubcoreMesh, core_axis_name="c", subcore_axis_name="s")


def _fill_rows(ref, nrows, ncols, value, dtype):
    v = jnp.full((16,), value, dtype)

    def body(i, _):
        for cc in range(ncols // 16):
            ref[i, pl.ds(cc * 16, 16)] = v
        return ()

    lax.fori_loop(0, nrows, body, ())


def _h_body(x_ref, w_ref, h_ref):
    h_ref[0] = jnp.dot(x_ref[...], w_ref[0],
                       preferred_element_type=jnp.float32)


def _make_h(x, W):
    nb = 2000
    return pl.pallas_call(
        _h_body,
        grid=(N // nb, R),
        in_specs=[
            pl.BlockSpec((nb, D), lambda n, r: (n, 0)),
            pl.BlockSpec((1, D, D), lambda n, r: (r, 0, 0)),
        ],
        out_specs=pl.BlockSpec((1, nb, D), lambda n, r: (r, n, 0)),
        out_shape=jax.ShapeDtypeStruct((R, N, D), jnp.float32),
    )(x, W)


def _count_body(zero_hbm, dst_hbm, typ_hbm, cpart_hbm,
                hist_v, dst_v, typ_v):
    c = lax.axis_index("c")
    s = lax.axis_index("s")
    wid = c * NS + s

    pltpu.sync_copy(zero_hbm, hist_v)

    ones = jnp.full((16,), 1.0, jnp.float32)

    def blk(j, _):
        off = wid * EPW + j * SB
        pltpu.sync_copy(dst_hbm.at[pl.ds(off, SB)], dst_v)
        pltpu.sync_copy(typ_hbm.at[pl.ds(off, SB)], typ_v)

        def grp(i, _):
            sl = pl.ds(i * 16, 16)
            cidx = dst_v[sl] * R + typ_v[sl]
            plsc.addupdate_scatter(hist_v, [cidx], ones)
            return ()

        lax.fori_loop(0, SB // 16, grp, ())
        return ()

    lax.fori_loop(0, NSBLK, blk, ())
    pltpu.sync_copy(hist_v, cpart_hbm.at[wid])


def _make_counts(dst, typ):
    kern = pl.kernel(
        _count_body,
        out_type=jax.ShapeDtypeStruct((NW, NRF), jnp.float32),
        mesh=_mesh(),
        scratch_types=[
            pltpu.VMEM((NRF,), jnp.float32),
            pltpu.VMEM((SB,), jnp.int32),
            pltpu.VMEM((SB,), jnp.int32),
        ],
        compiler_params=pltpu.CompilerParams(needs_layout_passes=False),
    )
    return kern(jnp.zeros((NRF,), jnp.float32), dst, typ)


def _inv_body(cp_ref, inv_ref):
    i = pl.program_id(0)

    @pl.when(i == 0)
    def _():
        inv_ref[...] = jnp.zeros_like(inv_ref)

    inv_ref[...] += jnp.sum(cp_ref[...], axis=0)

    @pl.when(i == pl.num_programs(0) - 1)
    def _():
        inv_ref[...] = 1.0 / jnp.maximum(inv_ref[...], 1.0)


def _make_inv(cpart):
    nw_b = 8
    cp3 = cpart.reshape(NW, NRF // 128, 128)
    inv2d = pl.pallas_call(
        _inv_body,
        grid=(NW // nw_b,),
        in_specs=[pl.BlockSpec((nw_b, NRF // 128, 128), lambda i: (i, 0, 0))],
        out_specs=pl.BlockSpec((NRF // 128, 128), lambda i: (0, 0)),
        out_shape=jax.ShapeDtypeStruct((NRF // 128, 128), jnp.float32),
    )(cp3)
    return inv2d.reshape(NRF)


def _scale_body(inv_hbm, src_hbm, dst_hbm, typ_hbm, gidx_hbm, sval_hbm,
                inv_v, src_v, dst_v, typ_v, gidx_v, sval_v):
    c = lax.axis_index("c")
    s = lax.axis_index("s")
    wid = c * NS + s

    pltpu.sync_copy(inv_hbm, inv_v)

    def blk(j, _):
        off = wid * EPW + j * SB
        pltpu.sync_copy(src_hbm.at[pl.ds(off, SB)], src_v)
        pltpu.sync_copy(dst_hbm.at[pl.ds(off, SB)], dst_v)
        pltpu.sync_copy(typ_hbm.at[pl.ds(off, SB)], typ_v)

        @plsc.parallel_loop(0, SB // 16, unroll=5)
        def _grp(i):
            sl = pl.ds(i * 16, 16)
            gidx_v[sl] = typ_v[sl] * N + src_v[sl]
            cix = dst_v[sl] * R + typ_v[sl]
            sval_v[sl] = plsc.load_gather(inv_v, [cix])
        pltpu.sync_copy(gidx_v, gidx_hbm.at[pl.ds(off, SB)])
        pltpu.sync_copy(sval_v, sval_hbm.at[pl.ds(off, SB)])
        return ()

    lax.fori_loop(0, NSBLK, blk, ())


def _make_scale(inv, src, dst, typ):
    kern = pl.kernel(
        _scale_body,
        out_type=(jax.ShapeDtypeStruct((E,), jnp.int32),
                  jax.ShapeDtypeStruct((E,), jnp.float32)),
        mesh=_mesh(),
        scratch_types=[
            pltpu.VMEM((NRF,), jnp.float32),
            pltpu.VMEM((SB,), jnp.int32),
            pltpu.VMEM((SB,), jnp.int32),
            pltpu.VMEM((SB,), jnp.int32),
            pltpu.VMEM((SB,), jnp.int32),
            pltpu.VMEM((SB,), jnp.float32),
        ],
        compiler_params=pltpu.CompilerParams(needs_layout_passes=False),
    )
    return kern(inv, src, dst, typ)


def _main_body(h_hbm, gidx_hbm, dst_hbm, sval_hbm, part_hbm,
               acc_sh, gidx_c, dst_c, sval_c, hbuf_a, hbuf_b,
               gsem_a, gsem_b, ssem_a, ssem_b):
    c = lax.axis_index("c")
    s = lax.axis_index("s")
    wid = c * NS + s

    _fill_rows(hbuf_a, 40, D, 0.0, jnp.float32)
    nz_s = jnp.where(s < 10, 16, 15)

    def zero_blk(k, _):
        pltpu.sync_copy(hbuf_a.at[pl.ds(0, 40)],
                        acc_sh.at[pl.ds((s + k * NS) * 40, 40)])
        return ()

    lax.fori_loop(0, nz_s, zero_blk, ())
    plsc.subcore_barrier()

    hbufs = (hbuf_a, hbuf_b)
    gsems = (gsem_a, gsem_b)
    ssems = (ssem_a, ssem_b)

    def scale_rows(hbuf, bb):
        @plsc.parallel_loop(0, EB, unroll=5)
        def _srow(e):
            svec = plsc.load_gather(
                sval_c, [jnp.full((16,), bb, jnp.int32),
                         jnp.full((16,), e, jnp.int32)])
            for cc in range(D // 16):
                csl = pl.ds(cc * 16, 16)
                hbuf[e, csl] = hbuf[e, csl] * svec

    def chunk(ch, _):
        rows = pl.ds(ch * CB, CB)
        pltpu.sync_copy(gidx_hbm.at[wid].at[rows], gidx_c)
        pltpu.sync_copy(dst_hbm.at[wid].at[rows], dst_c)
        pltpu.sync_copy(sval_hbm.at[wid].at[rows], sval_c)

        pltpu.async_copy(h_hbm.at[gidx_c.at[0]], hbufs[0], gsems[0])
        for b in range(CB):
            p = b % 2
            q = 1 - p
            if b + 1 < CB:
                if b >= 1:
                    pltpu.make_async_copy(hbufs[q], acc_sh.at[dst_c.at[b - 1]],
                                          ssems[q]).wait()
                pltpu.async_copy(h_hbm.at[gidx_c.at[b + 1]], hbufs[q],
                                 gsems[q])
            pltpu.make_async_copy(h_hbm.at[gidx_c.at[b]], hbufs[p],
                                  gsems[p]).wait()
            scale_rows(hbufs[p], b)
            if b + 1 < CB:
                pltpu.async_copy(hbufs[p], acc_sh.at[dst_c.at[b]], ssems[p],
                                 add=True)
            else:
                pltpu.sync_copy(hbufs[p], acc_sh.at[dst_c.at[b]], add=True)
        pltpu.make_async_copy(hbufs[CB % 2], acc_sh.at[dst_c.at[CB - 2]],
                              ssems[CB % 2]).wait()
        return ()

    lax.fori_loop(0, NBLK // CB, chunk, ())
    plsc.subcore_barrier()

    def out_blk(k, _):
        rows = pl.ds((s + k * NS) * 40, 40)
        pltpu.sync_copy(acc_sh.at[rows], hbuf_b.at[pl.ds(0, 40)])
        pltpu.sync_copy(hbuf_b.at[pl.ds(0, 40)], part_hbm.at[c].at[rows])
        return ()

    lax.fori_loop(0, nz_s, out_blk, ())


def _make_parts(h2, gidx, dst, sval):
    kern = pl.kernel(
        _main_body,
        out_type=jax.ShapeDtypeStruct((NC, N, D), jnp.float32),
        mesh=_mesh(),
        scratch_types=[
            pltpu.VMEM_SHARED((N, D), jnp.float32),
            pltpu.VMEM((CB, EB), jnp.int32),
            pltpu.VMEM((CB, EB), jnp.int32),
            pltpu.VMEM((CB, EB), jnp.float32),
            pltpu.VMEM((EB, D), jnp.float32),
            pltpu.VMEM((EB, D), jnp.float32),
            pltpu.SemaphoreType.DMA,
            pltpu.SemaphoreType.DMA,
            pltpu.SemaphoreType.DMA,
            pltpu.SemaphoreType.DMA,
        ],
        compiler_params=pltpu.CompilerParams(needs_layout_passes=False),
    )
    g3 = gidx.reshape(NW, NBLK, EB)
    d3 = dst.reshape(NW, NBLK, EB)
    v3 = sval.reshape(NW, NBLK, EB)
    return kern(h2, g3, d3, v3)


def _final_body(x_ref, root_ref, bias_ref, part_ref, out_ref):
    acc = jnp.dot(x_ref[...], root_ref[...],
                  preferred_element_type=jnp.float32)
    acc = acc + bias_ref[...] + part_ref[0] + part_ref[1]
    out_ref[...] = jnp.maximum(acc, 0.0)


def _make_out(x, root, bias, part):
    nb = 2000
    return pl.pallas_call(
        _final_body,
        grid=(N // nb,),
        in_specs=[
            pl.BlockSpec((nb, D), lambda n: (n, 0)),
            pl.BlockSpec((D, D), lambda n: (0, 0)),
            pl.BlockSpec((1, D), lambda n: (0, 0)),
            pl.BlockSpec((NC, nb, D), lambda n: (0, n, 0)),
        ],
        out_specs=pl.BlockSpec((nb, D), lambda n: (n, 0)),
        out_shape=jax.ShapeDtypeStruct((N, D), jnp.float32),
    )(x, root, bias.reshape(1, D), part)


@jax.jit
def kernel(x, edge_index, edge_type, W, root, bias):
    src = edge_index[0]
    dst = edge_index[1]
    typ = edge_type

    h = _make_h(x, W)
    h2 = h.reshape(R * N, D)
    cpart = _make_counts(dst, typ)
    inv = _make_inv(cpart)
    gidx, sval = _make_scale(inv, src, dst, typ)
    part = _make_parts(h2, gidx, dst, sval)
    return _make_out(x, root, bias, part)
